# Initial kernel scaffold; baseline (speedup 1.0000x reference)
import jax
import jax.numpy as jnp
from jax.experimental import pallas as pl


def kernel(x, edge_index, W1, b1, W2, b2, Wp1, bp1, Wp2, bp2, prototypes):
    raise NotImplementedError("write your pallas kernel here")


# speedup vs baseline: 6.8572x; 6.8572x over previous
import functools

import jax
import jax.numpy as jnp
from jax import lax
from jax.experimental import pallas as pl
from jax.experimental.pallas import tpu as pltpu
from jax.experimental.pallas import tpu_sc as plsc

N_NODES = 10000
N_PAD = 10240
N_EDGES = 160000
D = 128
CHUNK = 128
N_CHUNKS = N_EDGES // CHUNK
NW = 32
ROWS_PER_TILE = N_PAD // 16
DEG_W = 16
TEMP = 0.1

_mesh = plsc.VectorSubcoreMesh(core_axis_name="c", subcore_axis_name="s")


@functools.partial(
    pl.kernel,
    out_type=jax.ShapeDtypeStruct((2, N_PAD, DEG_W), jnp.float32),
    mesh=_mesh,
    scratch_types=[
        pltpu.VMEM((CHUNK,), jnp.int32),
        pltpu.VMEM((CHUNK, DEG_W), jnp.float32),
        pltpu.VMEM((CHUNK, DEG_W), jnp.float32),
        pltpu.VMEM_SHARED((N_PAD, DEG_W), jnp.float32),
    ],
)
def _deg_pass(dst_hbm, out_hbm, didx, ones, zeros, acc):
    c = lax.axis_index("c")
    s = lax.axis_index("s")
    wid = s * 2 + c

    def fill(i, _):
        ones[i, :] = jnp.ones((DEG_W,), jnp.float32)
        zeros[i, :] = jnp.zeros((DEG_W,), jnp.float32)
        return 0

    lax.fori_loop(0, CHUNK, fill, 0)
    for k in range(ROWS_PER_TILE // CHUNK):
        pltpu.sync_copy(zeros, acc.at[pl.ds(s * ROWS_PER_TILE + k * CHUNK, CHUNK)])
    plsc.subcore_barrier()

    n_i = (N_CHUNKS - wid + NW - 1) // NW

    def body(i, _):
        ch = wid + i * NW
        pltpu.sync_copy(dst_hbm.at[ch], didx)
        pltpu.sync_copy(ones, acc.at[didx], add=True)
        return 0

    lax.fori_loop(0, n_i, body, 0)
    plsc.subcore_barrier()
    pltpu.sync_copy(
        acc.at[pl.ds(s * ROWS_PER_TILE, ROWS_PER_TILE)],
        out_hbm.at[c, pl.ds(s * ROWS_PER_TILE, ROWS_PER_TILE)],
    )


@functools.partial(
    pl.kernel,
    out_type=jax.ShapeDtypeStruct((2, N_PAD, D), jnp.float32),
    mesh=_mesh,
    scratch_types=[
        pltpu.VMEM((CHUNK,), jnp.int32),
        pltpu.VMEM((CHUNK,), jnp.int32),
        pltpu.VMEM((CHUNK, D), jnp.float32),
        pltpu.VMEM((CHUNK, D), jnp.float32),
        pltpu.VMEM_SHARED((N_PAD, D), jnp.float32),
        pltpu.SemaphoreType.DMA,
    ],
)
def _scatter_pass(g_hbm, src_hbm, dst_hbm, out_hbm, sidx, didx, buf, zeros, acc, gsem):
    c = lax.axis_index("c")
    s = lax.axis_index("s")
    wid = s * 2 + c

    z16 = jnp.zeros((16,), jnp.float32)

    def zfill(i, _):
        zeros[i // 8, pl.ds((i % 8) * 16, 16)] = z16
        return 0

    lax.fori_loop(0, CHUNK * 8, zfill, 0)
    for k in range(ROWS_PER_TILE // CHUNK):
        pltpu.sync_copy(zeros, acc.at[pl.ds(s * ROWS_PER_TILE + k * CHUNK, CHUNK)])
    plsc.subcore_barrier()

    n_i = (N_CHUNKS - wid + NW - 1) // NW

    def body(i, _):
        ch = wid + i * NW
        pltpu.sync_copy(src_hbm.at[ch], sidx)
        pltpu.sync_copy(dst_hbm.at[ch], didx)
        pltpu.async_copy(g_hbm.at[sidx], buf, gsem).wait()
        pltpu.sync_copy(buf, acc.at[didx], add=True)
        return 0

    lax.fori_loop(0, n_i, body, 0)
    plsc.subcore_barrier()
    pltpu.sync_copy(
        acc.at[pl.ds(s * ROWS_PER_TILE, ROWS_PER_TILE)],
        out_hbm.at[c, pl.ds(s * ROWS_PER_TILE, ROWS_PER_TILE)],
    )


@functools.partial(
    pl.kernel,
    out_type=jax.ShapeDtypeStruct((N_EDGES, D), jnp.float32),
    mesh=_mesh,
    scratch_types=[
        pltpu.VMEM((CHUNK,), jnp.int32),
        pltpu.VMEM((CHUNK,), jnp.int32),
        pltpu.VMEM((CHUNK, D), jnp.float32),
        pltpu.VMEM((CHUNK, D), jnp.float32),
        pltpu.VMEM((CHUNK,), jnp.int32),
        pltpu.VMEM_SHARED((16 * CHUNK, D), jnp.float32),
        pltpu.SemaphoreType.DMA,
        pltpu.SemaphoreType.DMA,
    ],
)
def _edge_combine(a_hbm, b_hbm, src_hbm, dst_hbm, u_hbm,
                  sidx, didx, bufa, bufb, idmap, stage, sema, semb):
    c = lax.axis_index("c")
    s = lax.axis_index("s")
    wid = s * 2 + c

    iota16 = lax.iota(jnp.int32, 16)
    for g in range(CHUNK // 16):
        idmap[pl.ds(g * 16, 16)] = iota16 + (s * CHUNK + g * 16)

    n_i = (N_CHUNKS - wid + NW - 1) // NW

    def body(i, _):
        ch = wid + i * NW
        pltpu.sync_copy(src_hbm.at[ch], sidx)
        pltpu.sync_copy(dst_hbm.at[ch], didx)
        cpa = pltpu.async_copy(a_hbm.at[sidx], bufa, sema)
        cpb = pltpu.async_copy(b_hbm.at[didx], bufb, semb)
        cpa.wait()
        pltpu.sync_copy(bufa, stage.at[pl.ds(s * CHUNK, CHUNK)])
        cpb.wait()
        pltpu.sync_copy(bufb, stage.at[idmap], add=True)
        pltpu.sync_copy(stage.at[pl.ds(s * CHUNK, CHUNK)],
                        u_hbm.at[pl.ds(ch * CHUNK, CHUNK)])
        return 0

    lax.fori_loop(0, n_i, body, 0)


def _dinv_col(degp_ref):
    deg = 1.0 + degp_ref[0, :, 0:1] + degp_ref[1, :, 0:1]
    return lax.rsqrt(deg)


def _tc_g0_body(x_ref, w1_ref, degp_ref, o_ref):
    h = jnp.dot(x_ref[...], w1_ref[...], preferred_element_type=jnp.float32)
    o_ref[...] = h * _dinv_col(degp_ref)


def _tc_mid_body(sp_ref, g0_ref, degp_ref, b1_ref, w2_ref, o_ref):
    dinv = _dinv_col(degp_ref)
    agg = sp_ref[0] + sp_ref[1] + g0_ref[...]
    h1 = jnp.maximum(dinv * agg + b1_ref[...], 0.0)
    o_ref[...] = jnp.dot(h1, w2_ref[...], preferred_element_type=jnp.float32) * dinv


def _tc_post_body(sp_ref, g1_ref, degp_ref, b2_ref, wt_ref, wb_ref, bp1_ref,
                  a_ref, b_ref):
    dinv = _dinv_col(degp_ref)
    h2 = dinv * (sp_ref[0] + sp_ref[1] + g1_ref[...]) + b2_ref[...]
    a_ref[...] = jnp.dot(h2, wt_ref[...], preferred_element_type=jnp.float32) + bp1_ref[...]
    b_ref[...] = jnp.dot(h2, wb_ref[...], preferred_element_type=jnp.float32)


E_BLK = 1280


def _tc_final_body(u_ref, q_ref, k_ref, o_ref):
    r = jnp.maximum(u_ref[...], 0.0)
    sv = jnp.dot(r, q_ref[...], preferred_element_type=jnp.float32)
    d = k_ref[...] - sv
    col = lax.broadcasted_iota(jnp.int32, (E_BLK, D), 1)
    big = jnp.float32(3.0e38)
    m0 = jnp.min(jnp.where(col < 3, d, big), axis=1, keepdims=True)
    m1 = jnp.min(jnp.where((col >= 3) & (col < 6), d, big), axis=1, keepdims=True)
    o_ref[...] = jax.nn.sigmoid((m0 - m1) / TEMP)


_NB = N_PAD // 128

_tc_g0 = pl.pallas_call(
    _tc_g0_body,
    grid=(_NB,),
    in_specs=[
        pl.BlockSpec((128, D), lambda i: (i, 0)),
        pl.BlockSpec((D, D), lambda i: (0, 0)),
        pl.BlockSpec((2, 128, DEG_W), lambda i: (0, i, 0)),
    ],
    out_specs=pl.BlockSpec((128, D), lambda i: (i, 0)),
    out_shape=jax.ShapeDtypeStruct((N_PAD, D), jnp.float32),
)

_tc_mid = pl.pallas_call(
    _tc_mid_body,
    grid=(_NB,),
    in_specs=[
        pl.BlockSpec((2, 128, D), lambda i: (0, i, 0)),
        pl.BlockSpec((128, D), lambda i: (i, 0)),
        pl.BlockSpec((2, 128, DEG_W), lambda i: (0, i, 0)),
        pl.BlockSpec((1, D), lambda i: (0, 0)),
        pl.BlockSpec((D, D), lambda i: (0, 0)),
    ],
    out_specs=pl.BlockSpec((128, D), lambda i: (i, 0)),
    out_shape=jax.ShapeDtypeStruct((N_PAD, D), jnp.float32),
)

_tc_post = pl.pallas_call(
    _tc_post_body,
    grid=(_NB,),
    in_specs=[
        pl.BlockSpec((2, 128, D), lambda i: (0, i, 0)),
        pl.BlockSpec((128, D), lambda i: (i, 0)),
        pl.BlockSpec((2, 128, DEG_W), lambda i: (0, i, 0)),
        pl.BlockSpec((1, D), lambda i: (0, 0)),
        pl.BlockSpec((D, D), lambda i: (0, 0)),
        pl.BlockSpec((D, D), lambda i: (0, 0)),
        pl.BlockSpec((1, D), lambda i: (0, 0)),
    ],
    out_specs=[
        pl.BlockSpec((128, D), lambda i: (i, 0)),
        pl.BlockSpec((128, D), lambda i: (i, 0)),
    ],
    out_shape=[
        jax.ShapeDtypeStruct((N_PAD, D), jnp.float32),
        jax.ShapeDtypeStruct((N_PAD, D), jnp.float32),
    ],
)

_tc_final = pl.pallas_call(
    _tc_final_body,
    grid=(N_EDGES // E_BLK,),
    in_specs=[
        pl.BlockSpec((E_BLK, D), lambda i: (i, 0)),
        pl.BlockSpec((D, D), lambda i: (0, 0)),
        pl.BlockSpec((1, D), lambda i: (0, 0)),
    ],
    out_specs=pl.BlockSpec((E_BLK, 1), lambda i: (i, 0)),
    out_shape=jax.ShapeDtypeStruct((N_EDGES, 1), jnp.float32),
)


def kernel(x, edge_index, W1, b1, W2, b2, Wp1, bp1, Wp2, bp2, prototypes):
    src2d = edge_index[0].reshape(N_CHUNKS, CHUNK)
    dst2d = edge_index[1].reshape(N_CHUNKS, CHUNK)
    x_pad = jnp.pad(x, ((0, N_PAD - N_NODES), (0, 0)))

    degp = _deg_pass(dst2d)
    g0 = _tc_g0(x_pad, W1, degp)
    s1 = _scatter_pass(g0, src2d, dst2d)
    g1 = _tc_mid(s1, g0, degp, b1.reshape(1, D), W2)
    s2 = _scatter_pass(g1, src2d, dst2d)
    ap, bp = _tc_post(s2, g1, degp, b2.reshape(1, D), Wp1[:D], Wp1[D:],
                      bp1.reshape(1, D))
    u = _edge_combine(ap, bp, src2d, dst2d)

    p6 = prototypes.reshape(6, D)
    q = 2.0 * (Wp2 @ p6.T)
    kv = jnp.sum(p6 * p6, axis=1) - 2.0 * (p6 @ bp2)
    qpad = jnp.zeros((D, D), jnp.float32).at[:, :6].set(q)
    kpad = jnp.zeros((1, D), jnp.float32).at[0, :6].set(kv)

    probs = _tc_final(u, qpad, kpad)
    return probs.reshape(N_EDGES)
